# Initial kernel scaffold; baseline (speedup 1.0000x reference)
import jax
import jax.numpy as jnp
from jax.experimental import pallas as pl


def kernel(INPUT, userBias, itemBias, userEmbed, itemEmbed, w0):
    raise NotImplementedError("write your pallas kernel here")


# speedup vs baseline: 1.0264x; 1.0264x over previous
import functools

import jax
import jax.numpy as jnp
from jax import lax
from jax.experimental import pallas as pl
from jax.experimental.pallas import tpu as pltpu
from jax.experimental.pallas import tpu_sc as plsc

BATCH = 16384

_INFO = plsc.get_sparse_core_info()
_NC = _INFO.num_cores
_NS = _INFO.num_subcores
_L = _INFO.num_lanes
_NW = _NC * _NS
_BPW = BATCH // _NW
_CHUNK = 128
_NCHUNK = _BPW // _CHUNK
_NSL = _BPW // _L

_mesh = plsc.VectorSubcoreMesh(core_axis_name="c", subcore_axis_name="s")


@functools.partial(
    pl.kernel,
    out_type=jax.ShapeDtypeStruct((BATCH,), jnp.float32),
    mesh=_mesh,
    compiler_params=pltpu.CompilerParams(needs_layout_passes=False),
    scratch_types=[
        pltpu.VMEM((2 * _BPW,), jnp.int32),
        pltpu.VMEM((_NCHUNK, _CHUNK), jnp.int32),
        pltpu.VMEM((_NCHUNK, _CHUNK), jnp.int32),
        pltpu.VMEM((_BPW,), jnp.float32),
        pltpu.VMEM((_BPW,), jnp.float32),
        pltpu.VMEM((_BPW,), jnp.float32),
        pltpu.VMEM((_BPW,), jnp.float32),
        pltpu.VMEM((_L,), jnp.float32),
        pltpu.VMEM((_BPW,), jnp.float32),
        pltpu.SemaphoreType.DMA,
    ],
)
def _fm_sc(inp_hbm, ub_hbm, ib_hbm, ue_hbm, ie_hbm, w0_hbm, out_hbm,
           inp_v, uidx_v, iidx_v, ub_v, ib_v, ue_v, ie_v, w0_v, out_v, sem):
    wid = lax.axis_index("s") * _NC + lax.axis_index("c")
    base = wid * _BPW

    pltpu.sync_copy(inp_hbm.at[pl.ds(2 * base, 2 * _BPW)], inp_v)
    pltpu.sync_copy(w0_hbm, w0_v)

    iota2 = lax.iota(jnp.int32, _L) * 2
    for j in range(_NSL):
        upos = iota2 + (2 * _L) * j
        u = plsc.load_gather(inp_v, [upos])
        it = plsc.load_gather(inp_v, [upos + 1])
        uidx_v[j // 8, pl.ds((j % 8) * _L, _L)] = u
        iidx_v[j // 8, pl.ds((j % 8) * _L, _L)] = it

    copies = []
    for j in range(_NCHUNK):
        sl = pl.ds(j * _CHUNK, _CHUNK)
        copies.append(pltpu.async_copy(ub_hbm.at[uidx_v.at[j]], ub_v.at[sl], sem))
        copies.append(pltpu.async_copy(ib_hbm.at[iidx_v.at[j]], ib_v.at[sl], sem))
        copies.append(pltpu.async_copy(ue_hbm.at[uidx_v.at[j]], ue_v.at[sl], sem))
        copies.append(pltpu.async_copy(ie_hbm.at[iidx_v.at[j]], ie_v.at[sl], sem))
    for c in copies:
        c.wait()

    w0r = w0_v[...]
    for j in range(_NSL):
        sl = pl.ds(j * _L, _L)
        out_v[sl] = w0r + ub_v[sl] + ib_v[sl] + ue_v[sl] * ie_v[sl]

    pltpu.sync_copy(out_v, out_hbm.at[pl.ds(base, _BPW)])


def kernel(INPUT, userBias, itemBias, userEmbed, itemEmbed, w0):
    out = _fm_sc(
        INPUT.astype(jnp.int32).reshape(-1),
        userBias.reshape(-1),
        itemBias.reshape(-1),
        userEmbed.reshape(-1),
        itemEmbed.reshape(-1),
        jnp.broadcast_to(w0.reshape(()), (_L,)),
    )
    return out.reshape(BATCH, 1)
